# baseline (speedup 1.0000x reference)
import functools

import jax
import jax.numpy as jnp
from jax import lax
from jax.experimental import pallas as pl
from jax.experimental.pallas import tpu as pltpu
from jax.experimental.pallas import tpu_sc as plsc

NC = 2
NS = 16
NW = NC * NS
L = 16
CHUNK = 128


def _transpose_tc(o):
    d, n = o.shape
    blk = 512
    grid = pl.cdiv(n, blk)

    def body(x_ref, o_ref):
        o_ref[...] = x_ref[...].T

    return pl.pallas_call(
        body,
        grid=(grid,),
        in_specs=[pl.BlockSpec((d, blk), lambda i: (0, i))],
        out_specs=pl.BlockSpec((blk, d), lambda i: (i, 0)),
        out_shape=jax.ShapeDtypeStruct((n, d), o.dtype),
    )(o)


def _sc_forward(doc_b, ctx_b, smp_b, pmat, wmat, outT):
    _, BW = doc_b.shape
    C = ctx_b.shape[1]
    S = smp_b.shape[1]
    D = pmat.shape[1]
    CW = C * CHUNK
    SW = S * CHUNK
    HB = BW // 2
    mesh = plsc.VectorSubcoreMesh(core_axis_name="c", subcore_axis_name="s")

    @functools.partial(
        pl.kernel,
        out_type=jax.ShapeDtypeStruct((NW, SW), jnp.float32),
        mesh=mesh,
        scratch_types=[
            pltpu.VMEM((BW,), jnp.int32),
            pltpu.VMEM((C, CHUNK), jnp.int32),
            pltpu.VMEM((S, CHUNK), jnp.int32),
            pltpu.VMEM((BW, D), jnp.float32),
            pltpu.VMEM((CW // 2, D), jnp.float32),
            pltpu.VMEM((BW, D), jnp.float32),
            pltpu.VMEM((SW,), jnp.float32),
            pltpu.SemaphoreType.DMA,
        ],
        compiler_params=pltpu.CompilerParams(
            needs_layout_passes=False, use_tc_tiling_on_sc=False
        ),
    )
    def k(doc_hbm, ctx_hbm, smp_hbm, p_hbm, w_hbm, ot_hbm, out_hbm,
          idoc, ictx, ismp, prow, rows, acc, res, sem):
        wid = lax.axis_index("c") * NS + lax.axis_index("s")

        pltpu.sync_copy(doc_hbm.at[wid], idoc)
        pltpu.sync_copy(ctx_hbm.at[wid], ictx)
        pltpu.sync_copy(smp_hbm.at[wid], ismp)

        pltpu.async_copy(p_hbm.at[idoc], prow, sem).wait()

        for h in range(2):
            cps = [
                pltpu.async_copy(
                    w_hbm.at[ictx.at[h * (C // 2) + j]],
                    rows.at[pl.ds(j * CHUNK, CHUNK)],
                    sem,
                )
                for j in range(C // 2)
            ]
            for cp in cps:
                cp.wait()

            @pl.loop(0, HB)
            def _(b2, h=h):
                b = h * HB + b2
                r0 = b2 * C
                for kk in range(D // L):
                    sl = pl.ds(kk * L, L)
                    v = prow[b, sl]
                    for c in range(C):
                        v = v + rows[r0 + c, sl]
                    acc[b, sl] = v

        sps = [
            pltpu.async_copy(
                ot_hbm.at[ismp.at[j]],
                rows.at[pl.ds(j * CHUNK, CHUNK)],
                sem,
            )
            for j in range(S)
        ]
        for cp in sps:
            cp.wait()

        iota = lax.iota(jnp.int32, L)

        @pl.loop(0, SW // L)
        def _(g):
            i_vec = iota + g * L
            b_vec = i_vec // S

            def dbody(t, r):
                for dd in range(4):
                    d = t * 4 + dd
                    d_vec = jnp.full((L,), d, dtype=jnp.int32)
                    a = plsc.load_gather(acc, [b_vec, d_vec])
                    o = plsc.load_gather(rows, [i_vec, d_vec])
                    r = r + a * o
                return r

            r = lax.fori_loop(0, D // 4, dbody, jnp.zeros((L,), jnp.float32))
            res[pl.ds(g * L, L)] = r

        pltpu.sync_copy(res, out_hbm.at[wid])

    return k(doc_b, ctx_b, smp_b, pmat, wmat, outT)


def kernel(doc_ids, context_ids, sample_ids, paragraph_matrix, word_matrix,
           outputs):
    B, = doc_ids.shape
    C = context_ids.shape[1]
    S = sample_ids.shape[1]
    BW = B // NW
    outT = _transpose_tc(outputs)
    doc_b = doc_ids.reshape(NW, BW)
    ctx_b = context_ids.reshape(NW, (BW * C) // CHUNK, CHUNK)
    smp_b = sample_ids.reshape(NW, (BW * S) // CHUNK, CHUNK)
    res = _sc_forward(doc_b, ctx_b, smp_b, paragraph_matrix, word_matrix, outT)
    return res.reshape(B, S)


# speedup vs baseline: 1.1920x; 1.1920x over previous
import functools

import jax
import jax.numpy as jnp
from jax import lax
from jax.experimental import pallas as pl
from jax.experimental.pallas import tpu as pltpu
from jax.experimental.pallas import tpu_sc as plsc

NC = 2
NS = 16
NW = NC * NS
L = 16


def _transpose_to_half(o):
    d, n = o.shape
    blk = 512
    grid = pl.cdiv(n, blk)

    def body(x_ref, o_ref):
        o_ref[...] = x_ref[...].T.reshape(blk // 2, 2 * d)

    return pl.pallas_call(
        body,
        grid=(grid,),
        in_specs=[pl.BlockSpec((d, blk), lambda i: (0, i))],
        out_specs=pl.BlockSpec((blk // 2, 2 * d), lambda i: (i, 0)),
        out_shape=jax.ShapeDtypeStruct((n // 2, 2 * d), o.dtype),
    )(o)


def _sc_forward(ids, pmat, w2, ot2, C, S):
    BW = 128
    D = pmat.shape[1]
    KI = ids.shape[1]
    CCH = 4
    SCH = 4
    mesh = plsc.VectorSubcoreMesh(core_axis_name="c", subcore_axis_name="s")

    @functools.partial(
        pl.kernel,
        out_type=jax.ShapeDtypeStruct((NW, 8, 128), jnp.float32),
        mesh=mesh,
        scratch_types=[
            pltpu.VMEM((KI, 128), jnp.int32),
            pltpu.VMEM((C, 128), jnp.int32),
            pltpu.VMEM((S, 128), jnp.int32),
            pltpu.VMEM((L, 8, D), jnp.float32),
            pltpu.VMEM((BW * D,), jnp.float32),
            pltpu.VMEM((CCH * 128, 128), jnp.float32),
            pltpu.VMEM((D, 128), jnp.float32),
            pltpu.VMEM((8, 128), jnp.float32),
            pltpu.SemaphoreType.DMA,
            pltpu.SemaphoreType.DMA,
        ],
        compiler_params=pltpu.CompilerParams(
            needs_layout_passes=False, use_tc_tiling_on_sc=True
        ),
    )
    def k(ids_hbm, p_hbm, w2_hbm, ot2_hbm, out_hbm,
          idsv, cidx, sidx, ptile, prow, rows, acc, res, sem, sem2):
        wid = lax.axis_index("c") * NS + lax.axis_index("s")
        iota = lax.iota(jnp.int32, L)

        pltpu.sync_copy(ids_hbm.at[wid], idsv)

        @pl.loop(0, BW // L)
        def _(bb):
            docv = idsv[0, pl.ds(bb * L, L)]
            cps = []
            for t in range(L):
                start = pl.multiple_of(lax.bitwise_and(docv[t], -8), 8)
                cps.append(
                    pltpu.async_copy(
                        p_hbm.at[pl.ds(start, 8)], ptile.at[t], sem
                    )
                )
            for cp in cps:
                cp.wait()
            b0 = bb * L
            rvec = lax.bitwise_and(idsv[0, pl.ds(b0, L)], 7)
            b64 = (iota + b0) * D

            @pl.loop(0, D)
            def _(d, rvec=rvec, b64=b64):
                dv = jnp.full((L,), d, dtype=jnp.int32)
                v = plsc.load_gather(ptile, [iota, rvec, dv])
                plsc.store_scatter(prow, [b64 + dv], v)

        @pl.loop(0, C)
        def _(c):
            @pl.loop(0, 128 // L)
            def _(j):
                w = idsv[1 + c, pl.ds(j * L, L)]
                cidx[c, pl.ds(j * L, L)] = lax.shift_right_logical(w, 1)

        @pl.loop(0, S)
        def _(sj):
            @pl.loop(0, 128 // L)
            def _(j):
                w = idsv[1 + C + sj, pl.ds(j * L, L)]
                sidx[sj, pl.ds(j * L, L)] = lax.shift_right_logical(w, 1)

        for cc in range(C // CCH):
            gps = [
                pltpu.async_copy(
                    w2_hbm.at[cidx.at[cc * CCH + cl]],
                    rows.at[pl.ds(cl * 128, 128)],
                    sem2,
                )
                for cl in range(CCH)
            ]
            for cp in gps:
                cp.wait()

            @pl.loop(0, BW // L)
            def _(bg, cc=cc):
                b0 = bg * L
                b64 = (iota + b0) * D
                rowvecs = []
                colbases = []
                for cl in range(CCH):
                    w = idsv[1 + cc * CCH + cl, pl.ds(b0, L)]
                    rowvecs.append(iota + (cl * 128 + b0))
                    colbases.append(lax.bitwise_and(w, 1) * D)

                @pl.loop(0, D)
                def _(d, cc=cc, b0=b0, b64=b64,
                      rowvecs=rowvecs, colbases=colbases):
                    dv = jnp.full((L,), d, dtype=jnp.int32)
                    if cc == 0:
                        v = plsc.load_gather(prow, [b64 + dv])
                    else:
                        v = acc[d, pl.ds(b0, L)]
                    for cl in range(CCH):
                        v = v + plsc.load_gather(
                            rows, [rowvecs[cl], colbases[cl] + dv]
                        )
                    acc[d, pl.ds(b0, L)] = v

        for sc in range(pl.cdiv(S, SCH)):
            nr = min(SCH, S - sc * SCH)
            sps = [
                pltpu.async_copy(
                    ot2_hbm.at[sidx.at[sc * SCH + j]],
                    rows.at[pl.ds(j * 128, 128)],
                    sem2,
                )
                for j in range(nr)
            ]
            for cp in sps:
                cp.wait()

            @pl.loop(0, nr * (128 // L))
            def _(gl, sc=sc):
                g = sc * SCH * (128 // L) + gl
                r8 = g // (128 // L)
                c8 = g - r8 * (128 // L)
                w = idsv[1 + C + r8, pl.ds(c8 * L, L)]
                colb = lax.bitwise_and(w, 1) * D
                ivec = iota + gl * L
                bvec = (iota + g * L) // S

                def dbody(t, r):
                    for dd in range(4):
                        d = t * 4 + dd
                        dv = jnp.full((L,), d, dtype=jnp.int32)
                        a = plsc.load_gather(acc, [dv, bvec])
                        o = plsc.load_gather(rows, [ivec, colb + dv])
                        r = r + a * o
                    return r

                r = lax.fori_loop(0, D // 4, dbody,
                                  jnp.zeros((L,), jnp.float32))
                res[r8, pl.ds(c8 * L, L)] = r

        pltpu.sync_copy(res, out_hbm.at[wid])

    return k(ids, pmat, w2, ot2)


def kernel(doc_ids, context_ids, sample_ids, paragraph_matrix, word_matrix,
           outputs):
    B, = doc_ids.shape
    C = context_ids.shape[1]
    S = sample_ids.shape[1]
    BW = B // NW
    w2 = word_matrix.reshape(word_matrix.shape[0] // 2, 128)
    ot2 = outputs.T.reshape(outputs.shape[1] // 2, 128)
    doc_b = doc_ids.reshape(NW, 1, BW)
    ctx_b = context_ids.reshape(NW, BW, C).transpose(0, 2, 1)
    smp_b = sample_ids.reshape(NW, S, BW)
    pad = jnp.zeros((NW, 32 - 1 - C - S, BW), jnp.int32)
    ids = jnp.concatenate([doc_b, ctx_b, smp_b, pad], axis=1)
    res = _sc_forward(ids, paragraph_matrix, w2, ot2, C, S)
    return res[:, : S * BW // 128, :].reshape(B, S)
